# Initial kernel scaffold; baseline (speedup 1.0000x reference)
import jax
import jax.numpy as jnp
from jax.experimental import pallas as pl


def kernel(drug_x, pro_x, net_adj, W1, b1, W2, b2, W3, b3, emb_table, conv_w, conv_b, Wfc, bfc, Wg1, Wg2, Wg3):
    raise NotImplementedError("write your pallas kernel here")


# speedup vs baseline: 4.9907x; 4.9907x over previous
import functools

import jax
import jax.numpy as jnp
from jax import lax
from jax.experimental import pallas as pl
from jax.experimental.pallas import tpu as pltpu
from jax.experimental.pallas import tpu_sc as plsc

F32 = jnp.float32
ND, NPRO, NTOT = 3584, 512, 4096
EMB, H1, H2 = 128, 64, 32
V, L, KW, OC, TT = 26, 1000, 8, 32, 121

NC_SC, NS_SC = 2, 16
NW = NC_SC * NS_SC
NP_W = NPRO // NW
J, CH = 8, 125


def _prep_body(emb_ref, wfc_ref, w5_ref, cbrow_ref, bfc_ref, u_ref, bias_ref, t_ref):
    for k in range(KW):
        ek = emb_ref[:, k:k + TT]
        for o in range(OC):
            wo = wfc_ref[pl.ds(o * TT, TT), :]
            t_ref[k, o] = jnp.dot(ek, wo, preferred_element_type=F32)
    for v in range(V):
        tv = t_ref[:, :, v, :].reshape(KW * OC, EMB)
        u_ref[v] = jnp.dot(w5_ref[...], tv, preferred_element_type=F32)
    bias_ref[...] = (jnp.dot(cbrow_ref[...], wfc_ref[...], preferred_element_type=F32)
                     + bfc_ref[...])


def _drug_body(x_ref, w1_ref, b1_ref, w2_ref, b2_ref, w3_ref, b3_ref, wg1_ref, y_ref):
    h = jnp.maximum(jnp.dot(x_ref[...], w1_ref[...], preferred_element_type=F32)
                    + b1_ref[...], 0.0)
    h = jnp.maximum(jnp.dot(h, w2_ref[...], preferred_element_type=F32)
                    + b2_ref[...], 0.0)
    h = jnp.maximum(jnp.dot(h, w3_ref[...], preferred_element_type=F32)
                    + b3_ref[...], 0.0)
    y_ref[...] = jnp.dot(h, wg1_ref[...], preferred_element_type=F32)


def _y1p_body(pe_ref, bias_ref, wg1_ref, y_ref):
    y_ref[...] = jnp.dot(pe_ref[...] + bias_ref[...], wg1_ref[...],
                         preferred_element_type=F32)


def _mm_body(a_ref, b_ref, o_ref):
    o_ref[...] = jnp.dot(a_ref[...], b_ref[...], preferred_element_type=F32)


def _aggr_body(adj_ref, y_ref, o_ref, *, relu):
    r = jnp.dot(adj_ref[...], y_ref[...], preferred_element_type=F32)
    o_ref[...] = jnp.maximum(r, 0.0) if relu else r


def _dec_body(a_ref, b_ref, o_ref):
    o_ref[...] = lax.dot_general(a_ref[...], b_ref[...],
                                 (((1,), (1,)), ((), ())),
                                 preferred_element_type=F32)


def _sc_body(u_hbm, idx_hbm, out_hbm, idx_v, rows_v, acc_v, sem):
    wid = lax.axis_index("s") * NC_SC + lax.axis_index("c")
    base = wid * NP_W
    pltpu.sync_copy(idx_hbm.at[pl.ds(base, NP_W)], idx_v)

    def pbody(p, carry):
        acc = (jnp.zeros((16,), F32),) * 8
        for j in range(J):
            pltpu.async_copy(u_hbm.at[idx_v.at[p, j]], rows_v, sem).wait()

            def rbody(i, a):
                return tuple(a[e] + rows_v[i, pl.ds(e * 16, 16)] for e in range(8))

            acc = lax.fori_loop(0, CH, rbody, acc)
        for e in range(8):
            acc_v[pl.ds(e * 16, 16)] = acc[e]
        pltpu.sync_copy(acc_v, out_hbm.at[base + p])
        return carry

    lax.fori_loop(0, NP_W, pbody, 0)


_sc_gather = functools.partial(
    pl.kernel,
    out_type=jax.ShapeDtypeStruct((NPRO, EMB), F32),
    mesh=plsc.VectorSubcoreMesh(core_axis_name="c", subcore_axis_name="s"),
    scratch_types=[
        pltpu.VMEM((NP_W, J, CH), jnp.int32),
        pltpu.VMEM((CH, EMB), F32),
        pltpu.VMEM((EMB,), F32),
        pltpu.SemaphoreType.DMA,
    ],
)(_sc_body)


def kernel(drug_x, pro_x, net_adj, W1, b1, W2, b2, W3, b3, emb_table,
           conv_w, conv_b, Wfc, bfc, Wg1, Wg2, Wg3):
    w5 = conv_w.transpose(1, 2, 0).reshape(L, KW * OC)
    cbrow = jnp.repeat(conv_b, TT)[None, :]
    b1r, b2r, b3r, bfcr = b1[None, :], b2[None, :], b3[None, :], bfc[None, :]

    u, bias = pl.pallas_call(
        _prep_body,
        out_shape=(jax.ShapeDtypeStruct((V, L, EMB), F32),
                   jax.ShapeDtypeStruct((1, EMB), F32)),
        scratch_shapes=[pltpu.VMEM((KW, OC, V, EMB), F32)],
    )(emb_table, Wfc, w5, cbrow, bfcr)
    u_g = u.reshape(V * L, EMB)

    idx = (pro_x * L + lax.broadcasted_iota(jnp.int32, (1, L), 1)).reshape(NPRO, J, CH)

    pro_raw = _sc_gather(u_g, idx)

    y1d = pl.pallas_call(
        _drug_body,
        out_shape=jax.ShapeDtypeStruct((ND, H1), F32),
    )(drug_x, W1, b1r, W2, b2r, W3, b3r, Wg1)

    y1p = pl.pallas_call(
        _y1p_body,
        out_shape=jax.ShapeDtypeStruct((NPRO, H1), F32),
    )(pro_raw, bias, Wg1)

    y1 = jnp.concatenate([y1d, y1p], axis=0)

    BR = 256
    grid = NTOT // BR
    aggr = lambda relu: pl.pallas_call(
        functools.partial(_aggr_body, relu=relu),
        grid=(grid,),
        in_specs=[pl.BlockSpec((BR, NTOT), lambda i: (i, 0)),
                  pl.BlockSpec((NTOT, H1), lambda i: (0, 0))],
        out_specs=pl.BlockSpec((BR, H1), lambda i: (i, 0)),
        out_shape=jax.ShapeDtypeStruct((NTOT, H1), F32),
    )
    h1 = aggr(True)(net_adj, y1)

    w23 = jnp.concatenate([Wg2, Wg3], axis=1)
    y23 = pl.pallas_call(
        _mm_body,
        out_shape=jax.ShapeDtypeStruct((NTOT, H1), F32),
    )(h1, w23)
    ml = aggr(False)(net_adj, y23)
    mu, logvar = ml[:, :H2], ml[:, H2:]

    BD = 512
    adj_rec = pl.pallas_call(
        _dec_body,
        grid=(NTOT // BD, NTOT // BD),
        in_specs=[pl.BlockSpec((BD, H2), lambda i, j: (i, 0)),
                  pl.BlockSpec((BD, H2), lambda i, j: (j, 0))],
        out_specs=pl.BlockSpec((BD, BD), lambda i, j: (i, j)),
        out_shape=jax.ShapeDtypeStruct((NTOT, NTOT), F32),
    )(mu, mu)

    return adj_rec, mu, logvar


# speedup vs baseline: 6.1290x; 1.2281x over previous
import functools

import jax
import jax.numpy as jnp
from jax import lax
from jax.experimental import pallas as pl
from jax.experimental.pallas import tpu as pltpu
from jax.experimental.pallas import tpu_sc as plsc

F32 = jnp.float32
ND, NPRO, NTOT = 3584, 512, 4096
EMB, H1, H2 = 128, 64, 32
V, L, KW, OC, TT = 26, 1000, 8, 32, 121

NC_SC, NS_SC = 2, 16
NW = NC_SC * NS_SC
NP_W = NPRO // NW
J, CH = 8, 125
G = EMB // 16
UNR = 5


def _prep_body(emb_ref, wfc_ref, w5_ref, cbrow_ref, bfc_ref,
               u_ref, bias_ref, t_ref):
    for k in range(KW):
        ek = emb_ref[:, k:k + TT]
        for o in range(OC):
            wo = wfc_ref[pl.ds(o * TT, TT), :]
            t_ref[k, o] = jnp.dot(ek, wo, preferred_element_type=F32)
    for v in range(V):
        tv = t_ref[:, :, v, :].reshape(KW * OC, EMB)
        u_ref[v] = jnp.dot(w5_ref[...], tv, preferred_element_type=F32)
    bias_ref[...] = (jnp.dot(cbrow_ref[...], wfc_ref[...], preferred_element_type=F32)
                     + bfc_ref[...])


def _drug_body(x_ref, w1_ref, b1_ref, w2_ref, b2_ref, w3_ref, b3_ref, wg1_ref, y_ref):
    h = jnp.maximum(jnp.dot(x_ref[...], w1_ref[...], preferred_element_type=F32)
                    + b1_ref[...], 0.0)
    h = jnp.maximum(jnp.dot(h, w2_ref[...], preferred_element_type=F32)
                    + b2_ref[...], 0.0)
    h = jnp.maximum(jnp.dot(h, w3_ref[...], preferred_element_type=F32)
                    + b3_ref[...], 0.0)
    y_ref[...] = jnp.dot(h, wg1_ref[...], preferred_element_type=F32)


def _y1p_body(pe_ref, bias_ref, wg1_ref, y_ref):
    y_ref[...] = jnp.dot(pe_ref[...] + bias_ref[...], wg1_ref[...],
                         preferred_element_type=F32)


def _mm_body(a_ref, b_ref, o_ref):
    o_ref[...] = jnp.dot(a_ref[...], b_ref[...], preferred_element_type=F32)


def _aggr_body(adj_ref, y_ref, o_ref, *, relu):
    r = jnp.dot(adj_ref[...], y_ref[...], preferred_element_type=F32)
    o_ref[...] = jnp.maximum(r, 0.0) if relu else r


def _aggr_fixup_body(pa_ref, adj_ref, y_ref, o_ref):
    r = jnp.dot(adj_ref[...], y_ref[...], preferred_element_type=F32)
    o_ref[...] = jnp.maximum(pa_ref[...] + r, 0.0)


def _dec_body(a_ref, b_ref, o_ref):
    o_ref[...] = lax.dot_general(a_ref[...], b_ref[...],
                                 (((1,), (1,)), ((), ())),
                                 preferred_element_type=F32)


def _sc_body(u_hbm, idx_hbm, out_hbm,
             idx_v, rows0, rows1, acc_v, sem0, sem1):
    wid = lax.axis_index("s") * NC_SC + lax.axis_index("c")
    base = wid * NP_W
    pltpu.sync_copy(idx_hbm.at[pl.ds(base, NP_W)], idx_v)
    rows = (rows0, rows1)
    sems = (sem0, sem1)
    pltpu.make_async_copy(u_hbm.at[idx_v.at[0, 0]], rows0, sem0).start()

    def pbody(p, carry):
        acc = (jnp.zeros((16,), F32),) * G
        for j in range(J):
            b = j % 2
            pltpu.make_async_copy(u_hbm.at[idx_v.at[p, j]], rows[b], sems[b]).wait()
            if j < J - 1:
                nb = (j + 1) % 2
                pltpu.make_async_copy(u_hbm.at[idx_v.at[p, j + 1]],
                                      rows[nb], sems[nb]).start()
            else:
                @pl.when(p + 1 < NP_W)
                def _():
                    pltpu.make_async_copy(u_hbm.at[idx_v.at[p + 1, 0]],
                                          rows0, sem0).start()
            r = rows[b]

            def rbody(i, a):
                i0 = i * UNR
                for u in range(UNR):
                    a = tuple(a[e] + r[i0 + u, pl.ds(e * 16, 16)] for e in range(G))
                return a

            acc = lax.fori_loop(0, CH // UNR, rbody, acc)
        for e in range(G):
            acc_v[pl.ds(e * 16, 16)] = acc[e]
        pltpu.sync_copy(acc_v, out_hbm.at[base + p])
        return carry

    lax.fori_loop(0, NP_W, pbody, 0)


_sc_gather = functools.partial(
    pl.kernel,
    out_type=jax.ShapeDtypeStruct((NPRO, EMB), F32),
    mesh=plsc.VectorSubcoreMesh(core_axis_name="c", subcore_axis_name="s"),
    scratch_types=[
        pltpu.VMEM((NP_W, J, CH), jnp.int32),
        pltpu.VMEM((CH, EMB), F32),
        pltpu.VMEM((CH, EMB), F32),
        pltpu.VMEM((EMB,), F32),
        pltpu.SemaphoreType.DMA,
        pltpu.SemaphoreType.DMA,
    ],
)(_sc_body)


def kernel(drug_x, pro_x, net_adj, W1, b1, W2, b2, W3, b3, emb_table,
           conv_w, conv_b, Wfc, bfc, Wg1, Wg2, Wg3):
    w5 = conv_w.transpose(1, 2, 0).reshape(L, KW * OC)
    cbrow = jnp.repeat(conv_b, TT)[None, :]
    b1r, b2r, b3r, bfcr = b1[None, :], b2[None, :], b3[None, :], bfc[None, :]

    u, bias = pl.pallas_call(
        _prep_body,
        out_shape=(jax.ShapeDtypeStruct((V, L, EMB), F32),
                   jax.ShapeDtypeStruct((1, EMB), F32)),
        scratch_shapes=[pltpu.VMEM((KW, OC, V, EMB), F32)],
    )(emb_table, Wfc, w5, cbrow, bfcr)
    u_g = u.reshape(V * L, EMB)

    idx = (pro_x * L + lax.broadcasted_iota(jnp.int32, (1, L), 1)).reshape(NPRO, J, CH)

    pro_raw = _sc_gather(u_g, idx)

    y1p = pl.pallas_call(
        _y1p_body,
        out_shape=jax.ShapeDtypeStruct((NPRO, H1), F32),
    )(pro_raw, bias, Wg1)

    y1d = pl.pallas_call(
        _drug_body,
        out_shape=jax.ShapeDtypeStruct((ND, H1), F32),
    )(drug_x, W1, b1r, W2, b2r, W3, b3r, Wg1)

    BR = 256
    grid = NTOT // BR
    part_a = pl.pallas_call(
        functools.partial(_aggr_body, relu=False),
        grid=(grid,),
        in_specs=[pl.BlockSpec((BR, ND), lambda i: (i, 0)),
                  pl.BlockSpec((ND, H1), lambda i: (0, 0))],
        out_specs=pl.BlockSpec((BR, H1), lambda i: (i, 0)),
        out_shape=jax.ShapeDtypeStruct((NTOT, H1), F32),
    )(net_adj, y1d)

    h1 = pl.pallas_call(
        _aggr_fixup_body,
        grid=(grid,),
        in_specs=[pl.BlockSpec((BR, H1), lambda i: (i, 0)),
                  pl.BlockSpec((BR, NPRO), lambda i: (i, ND // NPRO)),
                  pl.BlockSpec((NPRO, H1), lambda i: (0, 0))],
        out_specs=pl.BlockSpec((BR, H1), lambda i: (i, 0)),
        out_shape=jax.ShapeDtypeStruct((NTOT, H1), F32),
    )(part_a, net_adj, y1p)

    w23 = jnp.concatenate([Wg2, Wg3], axis=1)
    y23 = pl.pallas_call(
        _mm_body,
        out_shape=jax.ShapeDtypeStruct((NTOT, H1), F32),
    )(h1, w23)
    ml = pl.pallas_call(
        functools.partial(_aggr_body, relu=False),
        grid=(grid,),
        in_specs=[pl.BlockSpec((BR, NTOT), lambda i: (i, 0)),
                  pl.BlockSpec((NTOT, H1), lambda i: (0, 0))],
        out_specs=pl.BlockSpec((BR, H1), lambda i: (i, 0)),
        out_shape=jax.ShapeDtypeStruct((NTOT, H1), F32),
    )(net_adj, y23)
    mu, logvar = ml[:, :H2], ml[:, H2:]

    BD = 512
    adj_rec = pl.pallas_call(
        _dec_body,
        grid=(NTOT // BD, NTOT // BD),
        in_specs=[pl.BlockSpec((BD, H2), lambda i, j: (i, 0)),
                  pl.BlockSpec((BD, H2), lambda i, j: (j, 0))],
        out_specs=pl.BlockSpec((BD, BD), lambda i, j: (i, j)),
        out_shape=jax.ShapeDtypeStruct((NTOT, NTOT), F32),
    )(mu, mu)

    return adj_rec, mu, logvar


# speedup vs baseline: 7.9709x; 1.3005x over previous
import functools

import jax
import jax.numpy as jnp
from jax import lax
from jax.experimental import pallas as pl
from jax.experimental.pallas import tpu as pltpu
from jax.experimental.pallas import tpu_sc as plsc

F32 = jnp.float32
ND, NPRO, NTOT = 3584, 512, 4096
EMB, H1, H2 = 128, 64, 32
V, L, KW, OC, TT = 26, 1000, 8, 32, 121

NC_SC, NS_SC = 2, 16
NW = NC_SC * NS_SC
NP_W = NPRO // NW
J, CH = 8, 125
G = EMB // 16
UNR = 5


def _prep_body(emb_ref, wfc_ref, w5_ref, cbrow_ref, bfc_ref,
               u_ref, bias_ref, t_ref):
    for k in range(KW):
        ek = emb_ref[:, k:k + TT]
        for o in range(OC):
            wo = wfc_ref[pl.ds(o * TT, TT), :]
            t_ref[k, o] = jnp.dot(ek, wo, preferred_element_type=F32)
    for v in range(V):
        tv = t_ref[:, :, v, :].reshape(KW * OC, EMB)
        u_ref[v] = jnp.dot(w5_ref[...], tv, preferred_element_type=F32)
    bias_ref[...] = (jnp.dot(cbrow_ref[...], wfc_ref[...], preferred_element_type=F32)
                     + bfc_ref[...])


def _drug_body(x_ref, w1_ref, b1_ref, w2_ref, b2_ref, w3_ref, b3_ref, wg1_ref, y_ref):
    h = jnp.maximum(jnp.dot(x_ref[...], w1_ref[...], preferred_element_type=F32)
                    + b1_ref[...], 0.0)
    h = jnp.maximum(jnp.dot(h, w2_ref[...], preferred_element_type=F32)
                    + b2_ref[...], 0.0)
    h = jnp.maximum(jnp.dot(h, w3_ref[...], preferred_element_type=F32)
                    + b3_ref[...], 0.0)
    y_ref[...] = jnp.dot(h, wg1_ref[...], preferred_element_type=F32)


def _aggr_body(adj_ref, y_ref, o_ref):
    o_ref[...] = jnp.dot(adj_ref[...], y_ref[...], preferred_element_type=F32)


def _aggrb_y23_body(pa_ref, adj_ref, praw_ref, bias_ref, wg1_ref, w23_ref, o_ref):
    y1p = jnp.dot(praw_ref[...] + bias_ref[...], wg1_ref[...],
                  preferred_element_type=F32)
    h = jnp.maximum(pa_ref[...] + jnp.dot(adj_ref[...], y1p,
                                          preferred_element_type=F32), 0.0)
    o_ref[...] = jnp.dot(h, w23_ref[...], preferred_element_type=F32)


def _aggr2_body(adj_ref, y_ref, mu_ref, lv_ref):
    r = jnp.dot(adj_ref[...], y_ref[...], preferred_element_type=F32)
    mu_ref[...] = r[:, :H2]
    lv_ref[...] = r[:, H2:]


def _dec_body(a_ref, b_ref, o_ref):
    o_ref[...] = lax.dot_general(a_ref[...], b_ref[...],
                                 (((1,), (1,)), ((), ())),
                                 preferred_element_type=F32)


NB = 4
NT = NP_W * J


def _sc_body(u_hbm, idx_hbm, out_hbm,
             idx_v, rows0, rows1, rows2, rows3, acc_v, sem0, sem1, sem2, sem3):
    wid = lax.axis_index("s") * NC_SC + lax.axis_index("c")
    base = wid * NP_W
    pltpu.sync_copy(idx_hbm.at[pl.ds(base, NP_W)], idx_v)
    rows = (rows0, rows1, rows2, rows3)
    sems = (sem0, sem1, sem2, sem3)

    def start(t, b):
        p, j = t // J, t % J
        pltpu.make_async_copy(u_hbm.at[idx_v.at[p, j]], rows[b], sems[b]).start()

    for t in range(NB - 1):
        start(t, t)

    def gbody(g, acc):
        for d in range(NB):
            t = g * NB + d
            p, j = t // J, t % J
            pltpu.make_async_copy(u_hbm.at[idx_v.at[p, j]], rows[d], sems[d]).wait()

            @pl.when(t + (NB - 1) < NT)
            def _():
                start(t + (NB - 1), (d + NB - 1) % NB)

            r = rows[d]

            def rbody(i, a):
                i0 = i * UNR
                for u in range(UNR):
                    a = tuple(a[e] + r[i0 + u, pl.ds(e * 16, 16)] for e in range(G))
                return a

            acc = lax.fori_loop(0, CH // UNR, rbody, acc)

            last = j == J - 1

            @pl.when(last)
            def _():
                for e in range(G):
                    acc_v[pl.ds(e * 16, 16)] = acc[e]
                pltpu.sync_copy(acc_v, out_hbm.at[base + p])

            acc = tuple(jnp.where(last, jnp.zeros((16,), F32), a) for a in acc)
        return acc

    lax.fori_loop(0, NT // NB, gbody, (jnp.zeros((16,), F32),) * G)


_sc_gather = functools.partial(
    pl.kernel,
    out_type=jax.ShapeDtypeStruct((NPRO, EMB), F32),
    mesh=plsc.VectorSubcoreMesh(core_axis_name="c", subcore_axis_name="s"),
    scratch_types=[
        pltpu.VMEM((NP_W, J, CH), jnp.int32),
        pltpu.VMEM((CH, EMB), F32),
        pltpu.VMEM((CH, EMB), F32),
        pltpu.VMEM((CH, EMB), F32),
        pltpu.VMEM((CH, EMB), F32),
        pltpu.VMEM((EMB,), F32),
        pltpu.SemaphoreType.DMA,
        pltpu.SemaphoreType.DMA,
        pltpu.SemaphoreType.DMA,
        pltpu.SemaphoreType.DMA,
    ],
)(_sc_body)


def kernel(drug_x, pro_x, net_adj, W1, b1, W2, b2, W3, b3, emb_table,
           conv_w, conv_b, Wfc, bfc, Wg1, Wg2, Wg3):
    w5 = conv_w.transpose(1, 2, 0).reshape(L, KW * OC)
    cbrow = jnp.repeat(conv_b, TT)[None, :]
    b1r, b2r, b3r, bfcr = b1[None, :], b2[None, :], b3[None, :], bfc[None, :]

    u, bias = pl.pallas_call(
        _prep_body,
        out_shape=(jax.ShapeDtypeStruct((V, L, EMB), F32),
                   jax.ShapeDtypeStruct((1, EMB), F32)),
        scratch_shapes=[pltpu.VMEM((KW, OC, V, EMB), F32)],
    )(emb_table, Wfc, w5, cbrow, bfcr)
    u_g = u.reshape(V * L, EMB)

    idx = (pro_x * L + lax.broadcasted_iota(jnp.int32, (1, L), 1)).reshape(NPRO, J, CH)

    pro_raw = _sc_gather(u_g, idx)

    y1d = pl.pallas_call(
        _drug_body,
        out_shape=jax.ShapeDtypeStruct((ND, H1), F32),
    )(drug_x, W1, b1r, W2, b2r, W3, b3r, Wg1)

    BR = 256
    grid = NTOT // BR
    part_a = pl.pallas_call(
        _aggr_body,
        grid=(grid,),
        in_specs=[pl.BlockSpec((BR, ND), lambda i: (i, 0)),
                  pl.BlockSpec((ND, H1), lambda i: (0, 0))],
        out_specs=pl.BlockSpec((BR, H1), lambda i: (i, 0)),
        out_shape=jax.ShapeDtypeStruct((NTOT, H1), F32),
    )(net_adj, y1d)

    w23 = jnp.concatenate([Wg2, Wg3], axis=1)
    y23 = pl.pallas_call(
        _aggrb_y23_body,
        grid=(grid,),
        in_specs=[pl.BlockSpec((BR, H1), lambda i: (i, 0)),
                  pl.BlockSpec((BR, NPRO), lambda i: (i, ND // NPRO)),
                  pl.BlockSpec((NPRO, EMB), lambda i: (0, 0)),
                  pl.BlockSpec((1, EMB), lambda i: (0, 0)),
                  pl.BlockSpec((EMB, H1), lambda i: (0, 0)),
                  pl.BlockSpec((H1, H1), lambda i: (0, 0))],
        out_specs=pl.BlockSpec((BR, H1), lambda i: (i, 0)),
        out_shape=jax.ShapeDtypeStruct((NTOT, H1), F32),
    )(part_a, net_adj, pro_raw, bias, Wg1, w23)

    mu, logvar = pl.pallas_call(
        _aggr2_body,
        grid=(grid,),
        in_specs=[pl.BlockSpec((BR, NTOT), lambda i: (i, 0)),
                  pl.BlockSpec((NTOT, H1), lambda i: (0, 0))],
        out_specs=[pl.BlockSpec((BR, H2), lambda i: (i, 0)),
                   pl.BlockSpec((BR, H2), lambda i: (i, 0))],
        out_shape=(jax.ShapeDtypeStruct((NTOT, H2), F32),
                   jax.ShapeDtypeStruct((NTOT, H2), F32)),
    )(net_adj, y23)

    BD = 512
    adj_rec = pl.pallas_call(
        _dec_body,
        grid=(NTOT // BD, NTOT // BD),
        in_specs=[pl.BlockSpec((BD, H2), lambda i, j: (i, 0)),
                  pl.BlockSpec((BD, H2), lambda i, j: (j, 0))],
        out_specs=pl.BlockSpec((BD, BD), lambda i, j: (i, j)),
        out_shape=jax.ShapeDtypeStruct((NTOT, NTOT), F32),
    )(mu, mu)

    return adj_rec, mu, logvar


# speedup vs baseline: 8.3120x; 1.0428x over previous
import functools

import jax
import jax.numpy as jnp
from jax import lax
from jax.experimental import pallas as pl
from jax.experimental.pallas import tpu as pltpu
from jax.experimental.pallas import tpu_sc as plsc

F32 = jnp.float32
ND, NPRO, NTOT = 3584, 512, 4096
EMB, H1, H2 = 128, 64, 32
V, L, KW, OC, TT = 26, 1000, 8, 32, 121

NC_SC, NS_SC = 2, 16
NW = NC_SC * NS_SC
G = EMB // 16
LP = 1024
CT = LP // NW
SB = 8
NSUB = CT // SB


def _prep_body(emb_ref, wfc_ref, w5_ref, cbrow_ref, bfc_ref,
               u_ref, bias_ref, t_ref):
    for k in range(KW):
        ek = emb_ref[:, k:k + TT]
        for o in range(OC):
            wo = wfc_ref[pl.ds(o * TT, TT), :]
            t_ref[k, o] = jnp.dot(ek, wo, preferred_element_type=F32)
    for v in range(V):
        tv = t_ref[:, :, v, :].reshape(KW * OC, EMB)
        u_ref[v, pl.ds(0, L), :] = jnp.dot(w5_ref[...], tv, preferred_element_type=F32)
        u_ref[v, pl.ds(L, LP - L), :] = jnp.zeros((LP - L, EMB), F32)
    bias_ref[...] = (jnp.dot(cbrow_ref[...], wfc_ref[...], preferred_element_type=F32)
                     + bfc_ref[...])


def _drug_body(x_ref, w1_ref, b1_ref, w2_ref, b2_ref, w3_ref, b3_ref, wg1_ref, y_ref):
    h = jnp.maximum(jnp.dot(x_ref[...], w1_ref[...], preferred_element_type=F32)
                    + b1_ref[...], 0.0)
    h = jnp.maximum(jnp.dot(h, w2_ref[...], preferred_element_type=F32)
                    + b2_ref[...], 0.0)
    h = jnp.maximum(jnp.dot(h, w3_ref[...], preferred_element_type=F32)
                    + b3_ref[...], 0.0)
    y_ref[...] = jnp.dot(h, wg1_ref[...], preferred_element_type=F32)


def _aggr_body(adj_ref, y_ref, o_ref):
    o_ref[...] = jnp.dot(adj_ref[...], y_ref[...], preferred_element_type=F32)


def _aggrb_y23_body(pa_ref, adj_ref, praw_ref, bias_ref, wg1_ref, w23_ref, o_ref):
    y1p = jnp.dot(praw_ref[0] + praw_ref[1] + bias_ref[...], wg1_ref[...],
                  preferred_element_type=F32)
    h = jnp.maximum(pa_ref[...] + jnp.dot(adj_ref[...], y1p,
                                          preferred_element_type=F32), 0.0)
    o_ref[...] = jnp.dot(h, w23_ref[...], preferred_element_type=F32)


def _aggr2_body(adj_ref, y_ref, mu_ref, lv_ref):
    r = jnp.dot(adj_ref[...], y_ref[...], preferred_element_type=F32)
    mu_ref[...] = r[:, :H2]
    lv_ref[...] = r[:, H2:]


def _dec_body(a_ref, b_ref, o_ref):
    o_ref[...] = lax.dot_general(a_ref[...], b_ref[...],
                                 (((1,), (1,)), ((), ())),
                                 preferred_element_type=F32)


def _sc_body(u_hbm, idx_hbm, out_hbm, u_sub, idx_v, acc_v, idxrow_v, shared):
    cid = lax.axis_index("c")
    sid = lax.axis_index("s")
    wid = sid * NC_SC + cid
    c0 = wid * CT
    pltpu.sync_copy(idx_hbm.at[pl.ds(c0, CT)], idx_v)

    def zbody(n16, carry):
        for g in range(G):
            acc_v[n16, pl.ds(g * 16, 16)] = jnp.zeros((16,), F32)
        return carry

    lax.fori_loop(0, NPRO, zbody, 0)

    def sbody(sub, carry):
        pltpu.sync_copy(u_hbm.at[:, pl.ds(c0 + sub * SB, SB), :], u_sub)

        def nbody(nb, carry2):
            n0 = nb * 16
            iv = [idx_v[sub * SB + c, pl.ds(n0, 16)] for c in range(SB)]
            for j in range(16):
                n = n0 + j
                acc = tuple(acc_v[n, pl.ds(g * 16, 16)] for g in range(G))
                for c in range(SB):
                    v = iv[c][j]
                    acc = tuple(acc[g] + u_sub[v, c, pl.ds(g * 16, 16)]
                                for g in range(G))
                for g in range(G):
                    acc_v[n, pl.ds(g * 16, 16)] = acc[g]
            return carry2

        lax.fori_loop(0, NPRO // 16, nbody, 0)
        return carry

    lax.fori_loop(0, NSUB, sbody, 0)

    for blk in range(NPRO // EMB):
        for g in range(G):
            idxrow_v[blk, pl.ds(g * 16, 16)] = (lax.iota(jnp.int32, 16)
                                                + (blk * EMB + g * 16))

    @pl.when(sid == 0)
    def _():
        pltpu.sync_copy(acc_v, shared)
    plsc.subcore_barrier()

    @pl.when(sid != 0)
    def _():
        for blk in range(NPRO // EMB):
            pltpu.sync_copy(acc_v.at[pl.ds(blk * EMB, EMB)],
                            shared.at[idxrow_v.at[blk]], add=True)
    plsc.subcore_barrier()

    @pl.when(sid == 0)
    def _():
        pltpu.sync_copy(shared, out_hbm.at[cid])


_sc_gather = functools.partial(
    pl.kernel,
    out_type=jax.ShapeDtypeStruct((NC_SC, NPRO, EMB), F32),
    mesh=plsc.VectorSubcoreMesh(core_axis_name="c", subcore_axis_name="s"),
    scratch_types=[
        pltpu.VMEM((V, SB, EMB), F32),
        pltpu.VMEM((CT, NPRO), jnp.int32),
        pltpu.VMEM((NPRO, EMB), F32),
        pltpu.VMEM((NPRO // EMB, EMB), jnp.int32),
        pltpu.VMEM_SHARED((NPRO, EMB), F32),
    ],
)(_sc_body)


def kernel(drug_x, pro_x, net_adj, W1, b1, W2, b2, W3, b3, emb_table,
           conv_w, conv_b, Wfc, bfc, Wg1, Wg2, Wg3):
    w5 = conv_w.transpose(1, 2, 0).reshape(L, KW * OC)
    cbrow = jnp.repeat(conv_b, TT)[None, :]
    b1r, b2r, b3r, bfcr = b1[None, :], b2[None, :], b3[None, :], bfc[None, :]

    u, bias = pl.pallas_call(
        _prep_body,
        out_shape=(jax.ShapeDtypeStruct((V, LP, EMB), F32),
                   jax.ShapeDtypeStruct((1, EMB), F32)),
        scratch_shapes=[pltpu.VMEM((KW, OC, V, EMB), F32)],
    )(emb_table, Wfc, w5, cbrow, bfcr)

    idx_t = jnp.pad(pro_x.transpose(), ((0, LP - L), (0, 0)))

    pro_raw = _sc_gather(u, idx_t)

    y1d = pl.pallas_call(
        _drug_body,
        out_shape=jax.ShapeDtypeStruct((ND, H1), F32),
    )(drug_x, W1, b1r, W2, b2r, W3, b3r, Wg1)

    BR = 256
    grid = NTOT // BR
    part_a = pl.pallas_call(
        _aggr_body,
        grid=(grid,),
        in_specs=[pl.BlockSpec((BR, ND), lambda i: (i, 0)),
                  pl.BlockSpec((ND, H1), lambda i: (0, 0))],
        out_specs=pl.BlockSpec((BR, H1), lambda i: (i, 0)),
        out_shape=jax.ShapeDtypeStruct((NTOT, H1), F32),
    )(net_adj, y1d)

    w23 = jnp.concatenate([Wg2, Wg3], axis=1)
    y23 = pl.pallas_call(
        _aggrb_y23_body,
        grid=(grid,),
        in_specs=[pl.BlockSpec((BR, H1), lambda i: (i, 0)),
                  pl.BlockSpec((BR, NPRO), lambda i: (i, ND // NPRO)),
                  pl.BlockSpec((NC_SC, NPRO, EMB), lambda i: (0, 0, 0)),
                  pl.BlockSpec((1, EMB), lambda i: (0, 0)),
                  pl.BlockSpec((EMB, H1), lambda i: (0, 0)),
                  pl.BlockSpec((H1, H1), lambda i: (0, 0))],
        out_specs=pl.BlockSpec((BR, H1), lambda i: (i, 0)),
        out_shape=jax.ShapeDtypeStruct((NTOT, H1), F32),
    )(part_a, net_adj, pro_raw, bias, Wg1, w23)

    mu, logvar = pl.pallas_call(
        _aggr2_body,
        grid=(grid,),
        in_specs=[pl.BlockSpec((BR, NTOT), lambda i: (i, 0)),
                  pl.BlockSpec((NTOT, H1), lambda i: (0, 0))],
        out_specs=[pl.BlockSpec((BR, H2), lambda i: (i, 0)),
                   pl.BlockSpec((BR, H2), lambda i: (i, 0))],
        out_shape=(jax.ShapeDtypeStruct((NTOT, H2), F32),
                   jax.ShapeDtypeStruct((NTOT, H2), F32)),
    )(net_adj, y23)

    BD = 512
    adj_rec = pl.pallas_call(
        _dec_body,
        grid=(NTOT // BD, NTOT // BD),
        in_specs=[pl.BlockSpec((BD, H2), lambda i, j: (i, 0)),
                  pl.BlockSpec((BD, H2), lambda i, j: (j, 0))],
        out_specs=pl.BlockSpec((BD, BD), lambda i, j: (i, j)),
        out_shape=jax.ShapeDtypeStruct((NTOT, NTOT), F32),
    )(mu, mu)

    return adj_rec, mu, logvar
